# baseline (speedup 1.0000x reference)
import functools

import jax
import jax.numpy as jnp
from jax import lax
from jax.experimental import pallas as pl
from jax.experimental.pallas import tpu as pltpu
from jax.experimental.pallas import tpu_sc as plsc

_P = 0.5
_SCALE = 1.0 / (1.0 - _P)
_NC = 2
_NS = 16
_NW = _NC * _NS
_L = 16
_BLK = 8192


@functools.cache
def _build(n):
  span = (n // _NW) // 8 * 8
  covered = span * _NW
  tail = n - covered
  nb = span // _BLK
  rem = span - nb * _BLK

  mesh = plsc.VectorSubcoreMesh(core_axis_name="c", subcore_axis_name="s")

  @functools.partial(
      pl.kernel,
      mesh=mesh,
      out_type=jax.ShapeDtypeStruct((n,), jnp.float32),
      scratch_types=[
          pltpu.VMEM((_BLK,), jnp.float32),
          pltpu.VMEM((_BLK,), jnp.float32),
      ],
  )
  def dropout_k(v_hbm, m_hbm, o_hbm, v_v, m_v):
    wid = lax.axis_index("s") * _NC + lax.axis_index("c")
    base = pl.multiple_of(wid * span, 8)

    def chunk(off, cnt):
      nvec = -(-cnt // _L)
      pltpu.sync_copy(v_hbm.at[pl.ds(off, cnt)], v_v.at[pl.ds(0, cnt)])
      pltpu.sync_copy(m_hbm.at[pl.ds(off, cnt)], m_v.at[pl.ds(0, cnt)])

      def inner(i, carry):
        sl = pl.ds(i * _L, _L)
        v = v_v[sl]
        m = m_v[sl]
        v_v[sl] = jnp.where(m >= _P, v * _SCALE, 0.0)
        return carry

      lax.fori_loop(0, nvec, inner, 0, unroll=8)
      pltpu.sync_copy(v_v.at[pl.ds(0, cnt)], o_hbm.at[pl.ds(off, cnt)])

    def blk_body(b, carry):
      chunk(pl.multiple_of(base + b * _BLK, 8), _BLK)
      return carry

    lax.fori_loop(0, nb, blk_body, 0)
    if rem:
      chunk(pl.multiple_of(base + nb * _BLK, 8), rem)
    if tail:
      @pl.when(wid == _NW - 1)
      def _tail():
        chunk(covered, tail)

  return dropout_k


def kernel(indices, values, mask_rand):
  del indices
  return _build(values.shape[0])(values, mask_rand)


# speedup vs baseline: 1.5504x; 1.5504x over previous
import functools

import jax
import jax.numpy as jnp
from jax import lax
from jax.experimental import pallas as pl
from jax.experimental.pallas import tpu as pltpu
from jax.experimental.pallas import tpu_sc as plsc

_P = 0.5
_SCALE = 1.0 / (1.0 - _P)
_NC = 2
_NS = 16
_NW = _NC * _NS
_L = 16
_BLK = 16384


@functools.cache
def _build(n):
  span = (n // _NW) // 8 * 8
  covered = span * _NW
  tail = n - covered
  nb = span // _BLK
  rem = span - nb * _BLK
  chunks = [(b * _BLK, _BLK) for b in range(nb)]
  if rem:
    chunks.append((nb * _BLK, rem))

  mesh = plsc.VectorSubcoreMesh(core_axis_name="c", subcore_axis_name="s")

  @functools.partial(
      pl.kernel,
      mesh=mesh,
      out_type=jax.ShapeDtypeStruct((n,), jnp.float32),
      scratch_types=[
          pltpu.VMEM((_BLK,), jnp.float32),
          pltpu.VMEM((_BLK,), jnp.float32),
          pltpu.VMEM((_BLK,), jnp.float32),
          pltpu.VMEM((_BLK,), jnp.float32),
          pltpu.VMEM((_BLK,), jnp.float32),
          pltpu.VMEM((_BLK,), jnp.float32),
          pltpu.SemaphoreType.DMA,
          pltpu.SemaphoreType.DMA,
          pltpu.SemaphoreType.DMA,
          pltpu.SemaphoreType.DMA,
          pltpu.SemaphoreType.DMA,
          pltpu.SemaphoreType.DMA,
      ],
  )
  def dropout_k(v_hbm, m_hbm, o_hbm, vb0, vb1, mb0, mb1, ob0, ob1,
                vs0, vs1, ms0, ms1, os0, os1):
    vbufs = (vb0, vb1)
    mbufs = (mb0, mb1)
    obufs = (ob0, ob1)
    v_sems = (vs0, vs1)
    m_sems = (ms0, ms1)
    o_sems = (os0, os1)
    wid = lax.axis_index("s") * _NC + lax.axis_index("c")
    base = wid * span

    def start_in(idx):
      p = idx % 2
      off = pl.multiple_of(base + chunks[idx][0], 8)
      cnt = chunks[idx][1]
      dv = pltpu.async_copy(v_hbm.at[pl.ds(off, cnt)],
                            vbufs[p].at[pl.ds(0, cnt)], v_sems[p])
      dm = pltpu.async_copy(m_hbm.at[pl.ds(off, cnt)],
                            mbufs[p].at[pl.ds(0, cnt)], m_sems[p])
      return dv, dm

    def compute(idx):
      p = idx % 2
      vb, mb, ob = vbufs[p], mbufs[p], obufs[p]
      nvec = -(-chunks[idx][1] // _L)

      @plsc.parallel_loop(0, nvec)
      def _body(i):
        sl = pl.ds(i * _L, _L)
        ob[sl] = jnp.where(mb[sl] >= _P, vb[sl] * _SCALE, 0.0)

    def start_out(idx):
      p = idx % 2
      off = pl.multiple_of(base + chunks[idx][0], 8)
      cnt = chunks[idx][1]
      return pltpu.async_copy(obufs[p].at[pl.ds(0, cnt)],
                              o_hbm.at[pl.ds(off, cnt)], o_sems[p])

    in_d = {0: start_in(0)}
    out_d = {}
    for idx in range(len(chunks)):
      if idx + 1 < len(chunks):
        in_d[idx + 1] = start_in(idx + 1)
      dv, dm = in_d.pop(idx)
      dv.wait()
      dm.wait()
      if idx >= 2:
        out_d.pop(idx - 2).wait()
      compute(idx)
      out_d[idx] = start_out(idx)
    for idx in sorted(out_d):
      out_d.pop(idx).wait()

    if tail:
      @pl.when(wid == _NW - 1)
      def _tail():
        nvec = -(-tail // _L)
        pltpu.sync_copy(v_hbm.at[pl.ds(covered, tail)],
                        vb0.at[pl.ds(0, tail)])
        pltpu.sync_copy(m_hbm.at[pl.ds(covered, tail)],
                        mb0.at[pl.ds(0, tail)])
        for i in range(nvec):
          sl = pl.ds(i * _L, _L)
          ob0[sl] = jnp.where(mb0[sl] >= _P, vb0[sl] * _SCALE, 0.0)
        pltpu.sync_copy(ob0.at[pl.ds(0, tail)],
                        o_hbm.at[pl.ds(covered, tail)])

  return dropout_k


def kernel(indices, values, mask_rand):
  del indices
  return _build(values.shape[0])(values, mask_rand)


# speedup vs baseline: 2.0608x; 1.3292x over previous
import functools

import jax
import jax.numpy as jnp
from jax import lax
from jax.experimental import pallas as pl
from jax.experimental.pallas import tpu as pltpu
from jax.experimental.pallas import tpu_sc as plsc

_P = 0.5
_SCALE = 1.0 / (1.0 - _P)
_NC = 2
_NS = 16
_NW = _NC * _NS
_L = 16
_BLK = 16384


@functools.cache
def _build(n):
  span = (n // _NW) // 8 * 8
  covered = span * _NW
  tail = n - covered
  nb = span // _BLK
  rem = span - nb * _BLK
  chunks = [(b * _BLK, _BLK) for b in range(nb)]
  if rem:
    chunks.append((nb * _BLK, rem))

  mesh = plsc.VectorSubcoreMesh(core_axis_name="c", subcore_axis_name="s")

  @functools.partial(
      pl.kernel,
      mesh=mesh,
      out_type=jax.ShapeDtypeStruct((n,), jnp.float32),
      scratch_types=[
          pltpu.VMEM((_BLK,), jnp.float32),
          pltpu.VMEM((_BLK,), jnp.float32),
          pltpu.VMEM((_BLK,), jnp.float32),
          pltpu.VMEM((_BLK,), jnp.float32),
          pltpu.VMEM((_BLK,), jnp.float32),
          pltpu.VMEM((_BLK,), jnp.float32),
          pltpu.SemaphoreType.DMA,
          pltpu.SemaphoreType.DMA,
          pltpu.SemaphoreType.DMA,
          pltpu.SemaphoreType.DMA,
          pltpu.SemaphoreType.DMA,
          pltpu.SemaphoreType.DMA,
      ],
  )
  def dropout_k(v_hbm, m_hbm, o_hbm, vb0, vb1, mb0, mb1, ob0, ob1,
                vs0, vs1, ms0, ms1, os0, os1):
    vbufs = (vb0, vb1)
    mbufs = (mb0, mb1)
    obufs = (ob0, ob1)
    v_sems = (vs0, vs1)
    m_sems = (ms0, ms1)
    o_sems = (os0, os1)
    wid = lax.axis_index("s") * _NC + lax.axis_index("c")
    base = wid * span

    def start_in(idx):
      p = idx % 2
      off = pl.multiple_of(base + chunks[idx][0], 8)
      cnt = chunks[idx][1]
      dv = pltpu.async_copy(v_hbm.at[pl.ds(off, cnt)],
                            vbufs[p].at[pl.ds(0, cnt)], v_sems[p])
      dm = pltpu.async_copy(m_hbm.at[pl.ds(off, cnt)],
                            mbufs[p].at[pl.ds(0, cnt)], m_sems[p])
      return dv, dm

    def compute(idx):
      p = idx % 2
      vb, mb, ob = vbufs[p], mbufs[p], obufs[p]
      nvec = -(-chunks[idx][1] // _L)

      @plsc.parallel_loop(0, nvec, unroll=8)
      def _body(i):
        sl = pl.ds(i * _L, _L)
        ob[sl] = jnp.where(mb[sl] >= _P, vb[sl] * _SCALE, 0.0)

    def start_out(idx):
      p = idx % 2
      off = pl.multiple_of(base + chunks[idx][0], 8)
      cnt = chunks[idx][1]
      return pltpu.async_copy(obufs[p].at[pl.ds(0, cnt)],
                              o_hbm.at[pl.ds(off, cnt)], o_sems[p])

    in_d = {0: start_in(0)}
    out_d = {}
    for idx in range(len(chunks)):
      if idx + 1 < len(chunks):
        in_d[idx + 1] = start_in(idx + 1)
      dv, dm = in_d.pop(idx)
      dv.wait()
      dm.wait()
      if idx >= 2:
        out_d.pop(idx - 2).wait()
      compute(idx)
      out_d[idx] = start_out(idx)
    for idx in sorted(out_d):
      out_d.pop(idx).wait()

    if tail:
      @pl.when(wid == _NW - 1)
      def _tail():
        nvec = -(-tail // _L)
        pltpu.sync_copy(v_hbm.at[pl.ds(covered, tail)],
                        vb0.at[pl.ds(0, tail)])
        pltpu.sync_copy(m_hbm.at[pl.ds(covered, tail)],
                        mb0.at[pl.ds(0, tail)])
        for i in range(nvec):
          sl = pl.ds(i * _L, _L)
          ob0[sl] = jnp.where(mb0[sl] >= _P, vb0[sl] * _SCALE, 0.0)
        pltpu.sync_copy(ob0.at[pl.ds(0, tail)],
                        o_hbm.at[pl.ds(covered, tail)])

  return dropout_k


def kernel(indices, values, mask_rand):
  del indices
  return _build(values.shape[0])(values, mask_rand)
